# baseline (speedup 1.0000x reference)
import functools

import jax
import jax.numpy as jnp
from jax import lax
from jax.experimental import pallas as pl
from jax.experimental.pallas import tpu as pltpu
from jax.experimental.pallas import tpu_sc as plsc

NUM_B = 256
NBINS = NUM_B * NUM_B
N_ELEM = 1024 * 1024
NC, NS, L = 2, 16, 16
NW = NC * NS
PER_W = N_ELEM // NW
WORDS_W = PER_W // 2
CHUNK = WORDS_W // 2
GRID = 4
MI_BLK = 32


def _minmax_tc(x, y):
    def body(x_ref, y_ref, o_ref):
        o_ref[0] = jnp.min(x_ref[...])
        o_ref[1] = jnp.max(x_ref[...])
        o_ref[2] = jnp.min(y_ref[...])
        o_ref[3] = jnp.max(y_ref[...])

    return pl.pallas_call(
        body,
        out_shape=jax.ShapeDtypeStruct((4,), jnp.float32),
        out_specs=pl.BlockSpec(memory_space=pltpu.SMEM),
    )(x, y)


def _bin_tc(x, y, mm):
    def body(mm_ref, x_ref, y_ref, o_ref):
        xn = (x_ref[...] - mm_ref[0]) / (mm_ref[1] - mm_ref[0] + 1e-08)
        yn = (y_ref[...] - mm_ref[2]) / (mm_ref[3] - mm_ref[2] + 1e-08)
        xb = jnp.clip(jnp.floor(xn * NUM_B).astype(jnp.int32), 0, NUM_B - 1)
        yb = jnp.clip(jnp.floor(yn * NUM_B).astype(jnp.int32), 0, NUM_B - 1)
        idx = xb * NUM_B + yb
        half = idx.shape[1] // 2
        o_ref[...] = idx[:, :half] | (idx[:, half:] << 16)

    return pl.pallas_call(
        body,
        grid=(GRID,),
        in_specs=[
            pl.BlockSpec(memory_space=pltpu.SMEM),
            pl.BlockSpec((1024, 1024 // GRID), lambda i: (0, i)),
            pl.BlockSpec((1024, 1024 // GRID), lambda i: (0, i)),
        ],
        out_shape=jax.ShapeDtypeStruct((4096, 128), jnp.int32),
        out_specs=pl.BlockSpec((4096 // GRID, 128), lambda i: (i, 0)),
    )(mm, x, y)


def _hist_sc(idx_flat, z):
    mesh = plsc.VectorSubcoreMesh(
        core_axis_name="c", subcore_axis_name="s", num_cores=NC, num_subcores=NS
    )
    cp = pltpu.CompilerParams(needs_layout_passes=False)

    @functools.partial(
        pl.kernel,
        out_type=jax.ShapeDtypeStruct((NW, NBINS // 128, 128), jnp.float32),
        mesh=mesh,
        compiler_params=cp,
        scratch_types=[
            pltpu.VMEM((NBINS // 128, 128), jnp.float32),
            pltpu.VMEM((WORDS_W,), jnp.int32),
            pltpu.SemaphoreType.DMA,
            pltpu.SemaphoreType.DMA,
            pltpu.SemaphoreType.DMA,
        ],
    )
    def k(idx_hbm, z_hbm, out_hbm, hist, ibuf, sem0, sem1, semz):
        c = lax.axis_index("c")
        s = lax.axis_index("s")
        wid = c * NS + s
        base = wid * WORDS_W
        sems = (sem0, sem1)

        zeros16 = jnp.zeros((L,), jnp.float32)
        ones16 = jnp.ones((L,), jnp.float32)

        def xfer(b):
            return pltpu.make_async_copy(
                idx_hbm.at[pl.ds(base + b * CHUNK, CHUNK)],
                ibuf.at[pl.ds(b * CHUNK, CHUNK)],
                sems[b],
            )

        zc = pltpu.make_async_copy(
            z_hbm.at[wid], hist.at[pl.ds(NBINS // 256, NBINS // 256)], semz
        )
        zc.start()
        xfer(0).start()
        xfer(1).start()

        @pl.loop(0, NBINS // 256)
        def _(r):
            for u in range(128 // L):
                hist[r, pl.ds(u * L, L)] = zeros16

        zc.wait()
        xfer(0).wait()
        xfer(1).wait()

        @pl.loop(0, WORDS_W, step=4 * L)
        def _(j):
            for u in range(4):
                v = ibuf[pl.ds(j + u * L, L)]
                for half in (
                    jnp.bitwise_and(v, 65535),
                    jax.lax.shift_right_logical(v, 16),
                ):
                    r = jax.lax.shift_right_logical(half, 7)
                    col = jnp.bitwise_and(half, 127)
                    plsc.addupdate_scatter(hist, [r, col], ones16)

        pltpu.sync_copy(hist, out_hbm.at[wid])

    return k(idx_flat, z)


def _mi_tc(h3):
    steps = NW // MI_BLK

    def body(h_ref, o_ref, acc):
        i = pl.program_id(0)
        part = jnp.sum(h_ref[...], axis=0)

        @pl.when(i == 0)
        def _():
            acc[...] = part

        @pl.when(i > 0)
        def _():
            acc[...] = acc[...] + part

        @pl.when(i == steps - 1)
        def _():
            h = acc[...].reshape(NUM_B, NUM_B)
            n = jnp.float32(N_ELEM)
            p_xy = h / n
            p_x = jnp.sum(p_xy, axis=1, keepdims=True)
            p_y = jnp.sum(p_xy, axis=0, keepdims=True)
            denom = p_x * p_y
            mask = (p_xy > 0) & (p_x > 0) & (p_y > 0)
            safe_ratio = jnp.where(mask, p_xy / jnp.where(mask, denom, 1.0), 1.0)
            terms = jnp.where(mask, p_xy * jnp.log(safe_ratio), 0.0)
            mi = jnp.sum(terms)
            h_x = -jnp.sum(p_x * jnp.log(p_x + 1e-08))
            h_y = -jnp.sum(p_y * jnp.log(p_y + 1e-08))
            mi = mi / (jnp.sqrt(h_x * h_y) + 1e-08)
            o_ref[0] = -mi

    out = pl.pallas_call(
        body,
        grid=(steps,),
        in_specs=[pl.BlockSpec((MI_BLK, 512, 128), lambda i: (i, 0, 0))],
        out_shape=jax.ShapeDtypeStruct((1,), jnp.float32),
        out_specs=pl.BlockSpec(memory_space=pltpu.SMEM),
        scratch_shapes=[pltpu.VMEM((512, 128), jnp.float32)],
    )(h3)
    return out[0]


def kernel(x, y):
    mm = _minmax_tc(x, y)
    idx = _bin_tc(x, y, mm)
    z = jnp.zeros((NW, NBINS // 256, 128), jnp.float32)
    hist3 = _hist_sc(idx.reshape(-1), z)
    return _mi_tc(hist3)


# speedup vs baseline: 1.0564x; 1.0564x over previous
import functools

import jax
import jax.numpy as jnp
from jax import lax
from jax.experimental import pallas as pl
from jax.experimental.pallas import tpu as pltpu
from jax.experimental.pallas import tpu_sc as plsc

NUM_B = 256
NBINS = NUM_B * NUM_B
N_ELEM = 1024 * 1024
NC, NS, L = 2, 16, 16
NW = NC * NS
PER_W = N_ELEM // NW
WORDS_W = PER_W // 2
CHUNK = WORDS_W // 2
GRID = 4
MI_BLK = 32


def _minmax_tc(x, y):
    def body(x_ref, y_ref, o_ref):
        o_ref[0] = jnp.min(x_ref[...])
        o_ref[1] = jnp.max(x_ref[...])
        o_ref[2] = jnp.min(y_ref[...])
        o_ref[3] = jnp.max(y_ref[...])

    return pl.pallas_call(
        body,
        out_shape=jax.ShapeDtypeStruct((4,), jnp.float32),
        out_specs=pl.BlockSpec(memory_space=pltpu.SMEM),
    )(x, y)


def _bin_tc(x, y, mm):
    def body(mm_ref, x_ref, y_ref, o_ref):
        xn = (x_ref[...] - mm_ref[0]) / (mm_ref[1] - mm_ref[0] + 1e-08)
        yn = (y_ref[...] - mm_ref[2]) / (mm_ref[3] - mm_ref[2] + 1e-08)
        xb = jnp.clip(jnp.floor(xn * NUM_B).astype(jnp.int32), 0, NUM_B - 1)
        yb = jnp.clip(jnp.floor(yn * NUM_B).astype(jnp.int32), 0, NUM_B - 1)
        idx = xb * NUM_B + yb
        half = idx.shape[1] // 2
        o_ref[...] = idx[:, :half] | (idx[:, half:] << 16)

    return pl.pallas_call(
        body,
        grid=(GRID,),
        in_specs=[
            pl.BlockSpec(memory_space=pltpu.SMEM),
            pl.BlockSpec((1024, 1024 // GRID), lambda i: (0, i)),
            pl.BlockSpec((1024, 1024 // GRID), lambda i: (0, i)),
        ],
        out_shape=jax.ShapeDtypeStruct((4096, 128), jnp.int32),
        out_specs=pl.BlockSpec((4096 // GRID, 128), lambda i: (i, 0)),
    )(mm, x, y)


def _hist_sc(idx_flat):
    mesh = plsc.VectorSubcoreMesh(
        core_axis_name="c", subcore_axis_name="s", num_cores=NC, num_subcores=NS
    )
    cp = pltpu.CompilerParams(needs_layout_passes=False)

    @functools.partial(
        pl.kernel,
        out_type=jax.ShapeDtypeStruct((NW, NBINS // 128, 128), jnp.float32),
        mesh=mesh,
        compiler_params=cp,
        scratch_types=[
            pltpu.VMEM((NBINS // 128, 128), jnp.float32),
            pltpu.VMEM((WORDS_W,), jnp.int32),
            pltpu.SemaphoreType.DMA,
            pltpu.SemaphoreType.DMA,
        ],
    )
    def k(idx_hbm, out_hbm, hist, ibuf, sem0, sem1):
        c = lax.axis_index("c")
        s = lax.axis_index("s")
        wid = c * NS + s
        base = wid * WORDS_W
        sems = (sem0, sem1)

        zeros16 = jnp.zeros((L,), jnp.float32)
        ones16 = jnp.ones((L,), jnp.float32)

        def xfer(b):
            return pltpu.make_async_copy(
                idx_hbm.at[pl.ds(base + b * CHUNK, CHUNK)],
                ibuf.at[pl.ds(b * CHUNK, CHUNK)],
                sems[b],
            )

        xfer(0).start()
        xfer(1).start()

        @pl.loop(0, NBINS // 128)
        def _(r):
            for u in range(128 // L):
                hist[r, pl.ds(u * L, L)] = zeros16

        xfer(0).wait()
        xfer(1).wait()

        @pl.loop(0, WORDS_W, step=4 * L)
        def _(j):
            for u in range(4):
                v = ibuf[pl.ds(j + u * L, L)]
                for half in (
                    jnp.bitwise_and(v, 65535),
                    jax.lax.shift_right_logical(v, 16),
                ):
                    r = jax.lax.shift_right_logical(half, 7)
                    col = jnp.bitwise_and(half, 127)
                    plsc.addupdate_scatter(hist, [r, col], ones16)

        pltpu.sync_copy(hist, out_hbm.at[wid])

    return k(idx_flat)


def _mi_tc(h3):
    steps = NW // MI_BLK

    def body(h_ref, o_ref, acc):
        i = pl.program_id(0)
        part = jnp.sum(h_ref[...], axis=0)

        @pl.when(i == 0)
        def _():
            acc[...] = part

        @pl.when(i > 0)
        def _():
            acc[...] = acc[...] + part

        @pl.when(i == steps - 1)
        def _():
            h = acc[...].reshape(NUM_B, NUM_B)
            n = jnp.float32(N_ELEM)
            p_xy = h / n
            p_x = jnp.sum(p_xy, axis=1, keepdims=True)
            p_y = jnp.sum(p_xy, axis=0, keepdims=True)
            denom = p_x * p_y
            mask = (p_xy > 0) & (p_x > 0) & (p_y > 0)
            safe_ratio = jnp.where(mask, p_xy / jnp.where(mask, denom, 1.0), 1.0)
            terms = jnp.where(mask, p_xy * jnp.log(safe_ratio), 0.0)
            mi = jnp.sum(terms)
            h_x = -jnp.sum(p_x * jnp.log(p_x + 1e-08))
            h_y = -jnp.sum(p_y * jnp.log(p_y + 1e-08))
            mi = mi / (jnp.sqrt(h_x * h_y) + 1e-08)
            o_ref[0] = -mi

    out = pl.pallas_call(
        body,
        grid=(steps,),
        in_specs=[pl.BlockSpec((MI_BLK, 512, 128), lambda i: (i, 0, 0))],
        out_shape=jax.ShapeDtypeStruct((1,), jnp.float32),
        out_specs=pl.BlockSpec(memory_space=pltpu.SMEM),
        scratch_shapes=[pltpu.VMEM((512, 128), jnp.float32)],
    )(h3)
    return out[0]


def kernel(x, y):
    mm = _minmax_tc(x, y)
    idx = _bin_tc(x, y, mm)
    hist3 = _hist_sc(idx.reshape(-1))
    return _mi_tc(hist3)
